# baseline (speedup 1.0000x reference)
import functools

import jax
import jax.numpy as jnp
from jax import lax
from jax.experimental import pallas as pl
from jax.experimental.pallas import tpu as pltpu
from jax.experimental.pallas import tpu_sc as plsc

B = 1024
D = 128
NUM_MEMORY = 100000
NUM_CLASSES = 1000
TEMP = 0.05
EPS = 1e-06

C_PAD = 1024
NC, NS = 2, 16
NW = NC * NS
RBLK = 128
CHUNK = 2 * RBLK
NCHUNK = NUM_MEMORY // CHUNK
EXTRA = NCHUNK % NW
BASEC = NCHUNK // NW
MAXJ = BASEC + 1
TAIL_START = NCHUNK * CHUNK
REM_START = TAIL_START + RBLK
REM = NUM_MEMORY - REM_START
LROWS = 2 * MAXJ
L2D = (NUM_MEMORY + RBLK - 1) // RBLK
TPW = B // NW
CROWS = C_PAD // NS
L2DP = 800
LSTAGE = 40


def _sc_body(feats, labels_h, labels2d, indexes_h, zf,
             ps_out, cnt_out, tgt_out,
             featA, featB, lab_all, cnt_v, idx_v, tgt_v, featr_v, labr_v,
             acc, semA, semB, semS, semT):
    c = lax.axis_index("c")
    s = lax.axis_index("s")
    w = s * NC + c
    schunk = BASEC * w + jnp.minimum(w, EXTRA)
    nchunk = jnp.where(w < EXTRA, BASEC + 1, BASEC)

    pltpu.sync_copy(indexes_h.at[pl.ds(w * TPW, TPW)], idx_v)
    pltpu.async_copy(labels_h.at[idx_v], tgt_v, semT)

    schunk2 = schunk * 2
    base = pl.multiple_of((schunk2 // 8) * 8, 8)
    off = schunk2 - base
    pltpu.sync_copy(labels2d.at[pl.ds(base, LSTAGE)], lab_all)

    pltpu.sync_copy(zf.at[pl.ds(s * CROWS, CROWS)], acc.at[pl.ds(s * CROWS, CROWS)])

    def zero_cnt(i, carry):
        cnt_v[pl.ds(i * 16, 16)] = jnp.zeros((16,), jnp.float32)
        return carry

    lax.fori_loop(0, C_PAD // 16, zero_cnt, 0)
    plsc.subcore_barrier()

    feat_bufs = (featA, featB)
    sems = (semA, semB)
    ones = jnp.full((16,), 1.0, jnp.float32)

    def start(j, b):
        @pl.when(j < nchunk)
        def _():
            st = pl.multiple_of((schunk + j) * CHUNK, CHUNK)
            pltpu.async_copy(feats.at[pl.ds(st, CHUNK)], feat_bufs[b], sems[b])

    def count_row(row):
        for i in range(RBLK // 16):
            lv = lab_all[off + row, pl.ds(i * 16, 16)]
            plsc.addupdate_scatter(cnt_v, [lv], ones)

    def process(j, b):
        start(j + 1, 1 - b)

        @pl.when(j < nchunk)
        def _():
            pltpu.make_async_copy(feats.at[pl.ds(0, CHUNK)],
                                  feat_bufs[b], sems[b]).wait()
            pltpu.async_copy(feat_bufs[b].at[pl.ds(0, RBLK)],
                             acc.at[lab_all.at[off + 2 * j]], semS, add=True)
            pltpu.sync_copy(feat_bufs[b].at[pl.ds(RBLK, RBLK)],
                            acc.at[lab_all.at[off + 2 * j + 1]], add=True)
            pltpu.make_async_copy(feats.at[pl.ds(0, RBLK)],
                                  feat_bufs[b].at[pl.ds(0, RBLK)], semS).wait()
            count_row(2 * j)
            count_row(2 * j + 1)

    start(0, 0)

    def pair_body(j2, carry):
        process(2 * j2, 0)
        process(2 * j2 + 1, 1)
        return carry

    lax.fori_loop(0, (MAXJ + 1) // 2, pair_body, 0)

    @pl.when(w == NW - 1)
    def _():
        pltpu.sync_copy(feats.at[pl.ds(TAIL_START, RBLK)], featA.at[pl.ds(0, RBLK)])
        pltpu.sync_copy(featA.at[pl.ds(0, RBLK)],
                        acc.at[lab_all.at[off + 2 * BASEC]], add=True)
        count_row(2 * BASEC)
        pltpu.sync_copy(feats.at[pl.ds(REM_START, REM)], featr_v)
        pltpu.sync_copy(labels_h.at[pl.ds(REM_START, REM)], labr_v)
        pltpu.sync_copy(featr_v, acc.at[labr_v], add=True)
        for i in range(REM // 16):
            lv = labr_v[pl.ds(i * 16, 16)]
            plsc.addupdate_scatter(cnt_v, [lv], ones)

    pltpu.sync_copy(cnt_v, cnt_out.at[pl.ds(w * C_PAD, C_PAD)])
    pltpu.make_async_copy(labels_h.at[pl.ds(0, TPW)], tgt_v, semT).wait()
    pltpu.sync_copy(tgt_v, tgt_out.at[pl.ds(w * TPW, TPW)])
    plsc.subcore_barrier()
    pltpu.sync_copy(acc.at[pl.ds(s * CROWS, CROWS)],
                    ps_out.at[c, pl.ds(s * CROWS, CROWS)])


@functools.cache
def _sc_segsum():
  return pl.kernel(
    _sc_body,
    out_type=(
        jax.ShapeDtypeStruct((NC, C_PAD, D), jnp.float32),
        jax.ShapeDtypeStruct((NW * C_PAD,), jnp.float32),
        jax.ShapeDtypeStruct((B,), jnp.int32),
    ),
    mesh=plsc.VectorSubcoreMesh(core_axis_name="c", subcore_axis_name="s",
                                num_cores=NC, num_subcores=NS),
    compiler_params=pltpu.CompilerParams(needs_layout_passes=False),
    scratch_types=[
        pltpu.VMEM((CHUNK, D), jnp.float32),
        pltpu.VMEM((CHUNK, D), jnp.float32),
        pltpu.VMEM((LSTAGE, RBLK), jnp.int32),
        pltpu.VMEM((C_PAD,), jnp.float32),
        pltpu.VMEM((TPW,), jnp.int32),
        pltpu.VMEM((TPW,), jnp.int32),
        pltpu.VMEM((REM, D), jnp.float32),
        pltpu.VMEM((REM,), jnp.int32),
        pltpu.VMEM_SHARED((C_PAD, D), jnp.float32),
        pltpu.SemaphoreType.DMA,
        pltpu.SemaphoreType.DMA,
        pltpu.SemaphoreType.DMA,
        pltpu.SemaphoreType.DMA,
    ],
  )


def _tc_body(x_ref, ps_ref, cnt_ref, tgt_ref, out_ref):
    x = x_ref[...]
    norm = jnp.sqrt(jnp.sum(x * x, axis=1, keepdims=True))
    x = x / jnp.maximum(norm, 1e-12)
    cf = ps_ref[0] + ps_ref[1]
    s = lax.dot_general(x, cf, dimension_numbers=(((1,), (1,)), ((), ())),
                        preferred_element_type=jnp.float32)
    nums = cnt_ref[pl.ds(0, C_PAD)][None, :]
    for i in range(1, NW):
        nums = nums + cnt_ref[pl.ds(i * C_PAD, C_PAD)][None, :]
    mask = (nums > 0).astype(jnp.float32)
    denom = TEMP * (mask * nums + (1.0 - mask))
    sim = s / denom
    exps = jnp.exp(sim) * mask
    sums = jnp.sum(exps, axis=1, keepdims=True) + EPS
    t = tgt_ref[...]
    cidx = lax.broadcasted_iota(jnp.int32, (B, C_PAD), 1)
    onehot = (cidx == t).astype(jnp.float32)
    picked = jnp.sum(exps * onehot, axis=1, keepdims=True)
    logp = jnp.log(picked / sums + EPS)
    out_ref[0, 0] = -jnp.mean(logp)


@functools.partial(jax.jit, static_argnames=("interpret",))
def _tc_loss(x, ps, cnt2, tgt2, interpret=False):
    return pl.pallas_call(
        _tc_body,
        out_shape=jax.ShapeDtypeStruct((1, 1), jnp.float32),
        out_specs=pl.BlockSpec(memory_space=pltpu.SMEM),
        interpret=interpret,
    )(x, ps, cnt2, tgt2)


def kernel(inputs, indexes, features, labels):
    labels2d = jnp.pad(labels, (0, L2DP * RBLK - NUM_MEMORY)).reshape(L2DP, RBLK)
    zf = jnp.zeros((C_PAD, D), jnp.float32)
    ps, cnt, targets = _sc_segsum()(features, labels, labels2d,
                                    indexes.astype(jnp.int32), zf)
    out = _tc_loss(inputs, ps, cnt, targets.reshape(B, 1))
    return out[0, 0]


# speedup vs baseline: 1.0288x; 1.0288x over previous
import functools

import jax
import jax.numpy as jnp
from jax import lax
from jax.experimental import pallas as pl
from jax.experimental.pallas import tpu as pltpu
from jax.experimental.pallas import tpu_sc as plsc

B = 1024
D = 128
NUM_MEMORY = 100000
NUM_CLASSES = 1000
TEMP = 0.05
EPS = 1e-06

C_PAD = 1024
NC, NS = 2, 16
NW = NC * NS
RBLK = 128
CHUNK = 2 * RBLK
NCHUNK = NUM_MEMORY // CHUNK
EXTRA = NCHUNK % NW
BASEC = NCHUNK // NW
MAXJ = BASEC + 1
TAIL_START = NCHUNK * CHUNK
REM_START = TAIL_START + RBLK
REM = NUM_MEMORY - REM_START
LROWS = 2 * MAXJ
L2D = (NUM_MEMORY + RBLK - 1) // RBLK
TPW = B // NW
CROWS = C_PAD // NS
L2DP = 800
LSTAGE = 40


def _sc_body(feats, labels_h, labels2d, indexes_h, zf,
             ps_out, cnt_out, tgt_out,
             featA, featB, lab_all, cnt_v, idx_v, tgt_v, featr_v, labr_v,
             acc, semA, semB, semS, semT):
    c = lax.axis_index("c")
    s = lax.axis_index("s")
    w = s * NC + c
    schunk = BASEC * w + jnp.minimum(w, EXTRA)
    nchunk = jnp.where(w < EXTRA, BASEC + 1, BASEC)

    pltpu.sync_copy(indexes_h.at[pl.ds(w * TPW, TPW)], idx_v)
    pltpu.async_copy(labels_h.at[idx_v], tgt_v, semT)

    schunk2 = schunk * 2
    base = pl.multiple_of((schunk2 // 8) * 8, 8)
    off = schunk2 - base
    pltpu.sync_copy(labels2d.at[pl.ds(base, LSTAGE)], lab_all)

    pltpu.sync_copy(zf.at[pl.ds(s * CROWS, CROWS)], acc.at[pl.ds(s * CROWS, CROWS)])

    def zero_cnt(i, carry):
        cnt_v[pl.ds(i * 16, 16)] = jnp.zeros((16,), jnp.float32)
        return carry

    lax.fori_loop(0, C_PAD // 16, zero_cnt, 0)
    plsc.subcore_barrier()

    feat_bufs = (featA, featB)
    sems = (semA, semB)
    ones = jnp.full((16,), 1.0, jnp.float32)

    def start(j, b):
        @pl.when(j < nchunk)
        def _():
            st = pl.multiple_of((schunk + j) * CHUNK, CHUNK)
            pltpu.async_copy(feats.at[pl.ds(st, CHUNK)], feat_bufs[b], sems[b])

    def count_row(row):
        for i in range(RBLK // 16):
            lv = lab_all[off + row, pl.ds(i * 16, 16)]
            plsc.addupdate_scatter(cnt_v, [lv], ones)

    def process(j, b):
        start(j + 1, 1 - b)

        @pl.when(j < nchunk)
        def _():
            pltpu.make_async_copy(feats.at[pl.ds(0, CHUNK)],
                                  feat_bufs[b], sems[b]).wait()
            pltpu.async_copy(feat_bufs[b].at[pl.ds(0, RBLK)],
                             acc.at[lab_all.at[off + 2 * j]], semS, add=True)
            pltpu.sync_copy(feat_bufs[b].at[pl.ds(RBLK, RBLK)],
                            acc.at[lab_all.at[off + 2 * j + 1]], add=True)
            pltpu.make_async_copy(feats.at[pl.ds(0, RBLK)],
                                  feat_bufs[b].at[pl.ds(0, RBLK)], semS).wait()
            count_row(2 * j)
            count_row(2 * j + 1)

    start(0, 0)

    def pair_body(j2, carry):
        process(2 * j2, 0)
        process(2 * j2 + 1, 1)
        return carry

    lax.fori_loop(0, (MAXJ + 1) // 2, pair_body, 0)

    @pl.when(w == NW - 1)
    def _():
        pltpu.sync_copy(feats.at[pl.ds(TAIL_START, RBLK)], featA.at[pl.ds(0, RBLK)])
        pltpu.sync_copy(featA.at[pl.ds(0, RBLK)],
                        acc.at[lab_all.at[off + 2 * BASEC]], add=True)
        count_row(2 * BASEC)
        pltpu.sync_copy(feats.at[pl.ds(REM_START, REM)], featr_v)
        pltpu.sync_copy(labels_h.at[pl.ds(REM_START, REM)], labr_v)
        pltpu.sync_copy(featr_v, acc.at[labr_v], add=True)
        for i in range(REM // 16):
            lv = labr_v[pl.ds(i * 16, 16)]
            plsc.addupdate_scatter(cnt_v, [lv], ones)

    pltpu.sync_copy(cnt_v, cnt_out.at[pl.ds(w * C_PAD, C_PAD)])
    pltpu.make_async_copy(labels_h.at[pl.ds(0, TPW)], tgt_v, semT).wait()
    pltpu.sync_copy(tgt_v, tgt_out.at[pl.ds(w * TPW, TPW)])
    plsc.subcore_barrier()
    pltpu.sync_copy(acc.at[pl.ds(s * CROWS, CROWS)],
                    ps_out.at[c, pl.ds(s * CROWS, CROWS)])


@functools.cache
def _sc_segsum():
  return pl.kernel(
    _sc_body,
    out_type=(
        jax.ShapeDtypeStruct((NC, C_PAD, D), jnp.float32),
        jax.ShapeDtypeStruct((NW * C_PAD,), jnp.float32),
        jax.ShapeDtypeStruct((B,), jnp.int32),
    ),
    mesh=plsc.VectorSubcoreMesh(core_axis_name="c", subcore_axis_name="s",
                                num_cores=NC, num_subcores=NS),
    compiler_params=pltpu.CompilerParams(needs_layout_passes=False),
    scratch_types=[
        pltpu.VMEM((CHUNK, D), jnp.float32),
        pltpu.VMEM((CHUNK, D), jnp.float32),
        pltpu.VMEM((LSTAGE, RBLK), jnp.int32),
        pltpu.VMEM((C_PAD,), jnp.float32),
        pltpu.VMEM((TPW,), jnp.int32),
        pltpu.VMEM((TPW,), jnp.int32),
        pltpu.VMEM((REM, D), jnp.float32),
        pltpu.VMEM((REM,), jnp.int32),
        pltpu.VMEM_SHARED((C_PAD, D), jnp.float32),
        pltpu.SemaphoreType.DMA,
        pltpu.SemaphoreType.DMA,
        pltpu.SemaphoreType.DMA,
        pltpu.SemaphoreType.DMA,
    ],
  )


def _tc_body(x_ref, ps_ref, cnt_ref, tgt_ref, out_ref):
    x = x_ref[...]
    norm = jnp.sqrt(jnp.sum(x * x, axis=1, keepdims=True))
    x = x / jnp.maximum(norm, 1e-12)
    cf = ps_ref[0] + ps_ref[1]
    s = lax.dot_general(x, cf, dimension_numbers=(((1,), (1,)), ((), ())),
                        preferred_element_type=jnp.float32)
    nums = cnt_ref[pl.ds(0, C_PAD)][None, :]
    for i in range(1, NW):
        nums = nums + cnt_ref[pl.ds(i * C_PAD, C_PAD)][None, :]
    mask = (nums > 0).astype(jnp.float32)
    denom = TEMP * (mask * nums + (1.0 - mask))
    sim = s / denom
    exps = jnp.exp(sim) * mask
    sums = jnp.sum(exps, axis=1, keepdims=True) + EPS
    t_row = tgt_ref[...][None, :].astype(jnp.float32)
    t = jnp.transpose(t_row)
    cidx = lax.broadcasted_iota(jnp.int32, (B, C_PAD), 1).astype(jnp.float32)
    onehot = (cidx == t).astype(jnp.float32)
    picked = jnp.sum(exps * onehot, axis=1, keepdims=True)
    logp = jnp.log(picked / sums + EPS)
    out_ref[0, 0] = -jnp.mean(logp)


@functools.partial(jax.jit, static_argnames=("interpret",))
def _tc_loss(x, ps, cnt2, tgt2, interpret=False):
    return pl.pallas_call(
        _tc_body,
        out_shape=jax.ShapeDtypeStruct((1, 1), jnp.float32),
        out_specs=pl.BlockSpec(memory_space=pltpu.SMEM),
        interpret=interpret,
    )(x, ps, cnt2, tgt2)


def kernel(inputs, indexes, features, labels):
    labels2d = jnp.pad(labels, (0, L2DP * RBLK - NUM_MEMORY)).reshape(L2DP, RBLK)
    zf = jnp.zeros((C_PAD, D), jnp.float32)
    ps, cnt, targets = _sc_segsum()(features, labels, labels2d,
                                    indexes.astype(jnp.int32), zf)
    out = _tc_loss(inputs, ps, cnt, targets)
    return out[0, 0]


# speedup vs baseline: 1.1073x; 1.0763x over previous
import functools

import jax
import jax.numpy as jnp
from jax import lax
from jax.experimental import pallas as pl
from jax.experimental.pallas import tpu as pltpu
from jax.experimental.pallas import tpu_sc as plsc

B = 1024
D = 128
NUM_MEMORY = 100000
NUM_CLASSES = 1000
TEMP = 0.05
EPS = 1e-06

C_PAD = 1024
NC, NS = 2, 16
NW = NC * NS
RBLK = 128
CHUNK = 2 * RBLK

SC_ROWS = 86016
NCHUNK = SC_ROWS // CHUNK
EXTRA = NCHUNK % NW
BASEC = NCHUNK // NW
MAXJ = BASEC + 1
TPW = B // NW
CROWS = C_PAD // NS
L2DP = 680
LSTAGE = 32

TC_ROWS = NUM_MEMORY - SC_ROWS
TCB = 1024
TC_STEPS = (TC_ROWS + TCB - 1) // TCB
TC_OFF = SC_ROWS // TCB


def _sc_body(feats, labels_h, labels2d, indexes_h, zf,
             ps_out, cnt_out, tgt_out,
             featA, featB, lab_all, cnt_v, idx_v, tgt_v,
             acc, semA, semB, semS, semT):
    c = lax.axis_index("c")
    s = lax.axis_index("s")
    w = s * NC + c
    schunk = BASEC * w + jnp.minimum(w, EXTRA)
    nchunk = jnp.where(w < EXTRA, BASEC + 1, BASEC)

    pltpu.sync_copy(indexes_h.at[pl.ds(w * TPW, TPW)], idx_v)
    pltpu.async_copy(labels_h.at[idx_v], tgt_v, semT)

    schunk2 = schunk * 2
    base = pl.multiple_of((schunk2 // 8) * 8, 8)
    off = schunk2 - base
    pltpu.sync_copy(labels2d.at[pl.ds(base, LSTAGE)], lab_all)

    pltpu.sync_copy(zf.at[pl.ds(s * CROWS, CROWS)], acc.at[pl.ds(s * CROWS, CROWS)])

    def zero_cnt(i, carry):
        cnt_v[pl.ds(i * 16, 16)] = jnp.zeros((16,), jnp.float32)
        return carry

    lax.fori_loop(0, C_PAD // 16, zero_cnt, 0)
    plsc.subcore_barrier()

    feat_bufs = (featA, featB)
    sems = (semA, semB)
    ones = jnp.full((16,), 1.0, jnp.float32)

    def start(j, b):
        @pl.when(j < nchunk)
        def _():
            st = pl.multiple_of((schunk + j) * CHUNK, CHUNK)
            pltpu.async_copy(feats.at[pl.ds(st, CHUNK)], feat_bufs[b], sems[b])

    def count_row(row):
        for i in range(RBLK // 16):
            lv = lab_all[off + row, pl.ds(i * 16, 16)]
            plsc.addupdate_scatter(cnt_v, [lv], ones)

    def process(j, b):
        start(j + 1, 1 - b)

        @pl.when(j < nchunk)
        def _():
            pltpu.make_async_copy(feats.at[pl.ds(0, CHUNK)],
                                  feat_bufs[b], sems[b]).wait()
            pltpu.async_copy(feat_bufs[b].at[pl.ds(0, RBLK)],
                             acc.at[lab_all.at[off + 2 * j]], semS, add=True)
            pltpu.sync_copy(feat_bufs[b].at[pl.ds(RBLK, RBLK)],
                            acc.at[lab_all.at[off + 2 * j + 1]], add=True)
            pltpu.make_async_copy(feats.at[pl.ds(0, RBLK)],
                                  feat_bufs[b].at[pl.ds(0, RBLK)], semS).wait()
            count_row(2 * j)
            count_row(2 * j + 1)

    start(0, 0)

    def pair_body(j2, carry):
        process(2 * j2, 0)
        process(2 * j2 + 1, 1)
        return carry

    lax.fori_loop(0, (MAXJ + 1) // 2, pair_body, 0)

    pltpu.sync_copy(cnt_v, cnt_out.at[pl.ds(w * C_PAD, C_PAD)])
    pltpu.make_async_copy(labels_h.at[pl.ds(0, TPW)], tgt_v, semT).wait()
    pltpu.sync_copy(tgt_v, tgt_out.at[pl.ds(w * TPW, TPW)])
    plsc.subcore_barrier()
    pltpu.sync_copy(acc.at[pl.ds(s * CROWS, CROWS)],
                    ps_out.at[c, pl.ds(s * CROWS, CROWS)])


@functools.cache
def _sc_segsum():
  return pl.kernel(
    _sc_body,
    out_type=(
        jax.ShapeDtypeStruct((NC, C_PAD, D), jnp.float32),
        jax.ShapeDtypeStruct((NW * C_PAD,), jnp.float32),
        jax.ShapeDtypeStruct((B,), jnp.int32),
    ),
    mesh=plsc.VectorSubcoreMesh(core_axis_name="c", subcore_axis_name="s",
                                num_cores=NC, num_subcores=NS),
    compiler_params=pltpu.CompilerParams(needs_layout_passes=False),
    scratch_types=[
        pltpu.VMEM((CHUNK, D), jnp.float32),
        pltpu.VMEM((CHUNK, D), jnp.float32),
        pltpu.VMEM((LSTAGE, RBLK), jnp.int32),
        pltpu.VMEM((C_PAD,), jnp.float32),
        pltpu.VMEM((TPW,), jnp.int32),
        pltpu.VMEM((TPW,), jnp.int32),
        pltpu.VMEM_SHARED((C_PAD, D), jnp.float32),
        pltpu.SemaphoreType.DMA,
        pltpu.SemaphoreType.DMA,
        pltpu.SemaphoreType.DMA,
        pltpu.SemaphoreType.DMA,
    ],
  )


def _tcseg_body(feat_ref, lab_ref, cf_ref, cnt_ref):
    j = pl.program_id(0)

    @pl.when(j == 0)
    def _():
        cf_ref[...] = jnp.zeros((C_PAD, D), jnp.float32)
        cnt_ref[...] = jnp.zeros((8, C_PAD), jnp.float32)

    lab_row = lab_ref[0, 0:1, :].astype(jnp.float32)
    lab_col = jnp.transpose(lab_row)
    cidx = lax.broadcasted_iota(jnp.int32, (TCB, C_PAD), 1).astype(jnp.float32)
    onehot = (cidx == lab_col).astype(jnp.float32)
    cf_ref[...] += lax.dot_general(onehot, feat_ref[...],
                                   dimension_numbers=(((0,), (0,)), ((), ())),
                                   preferred_element_type=jnp.float32)
    cnt_ref[...] += jnp.broadcast_to(
        jnp.sum(onehot, axis=0, keepdims=True), (8, C_PAD))


@functools.partial(jax.jit, static_argnames=("interpret",))
def _tc_segsum(features, labels_tc, interpret=False):
    return pl.pallas_call(
        _tcseg_body,
        grid=(TC_STEPS,),
        in_specs=[
            pl.BlockSpec((TCB, D), lambda j: (TC_OFF + j, 0)),
            pl.BlockSpec((1, 1, TCB), lambda j: (j, 0, 0)),
        ],
        out_specs=[
            pl.BlockSpec((C_PAD, D), lambda j: (0, 0)),
            pl.BlockSpec((8, C_PAD), lambda j: (0, 0)),
        ],
        out_shape=[
            jax.ShapeDtypeStruct((C_PAD, D), jnp.float32),
            jax.ShapeDtypeStruct((8, C_PAD), jnp.float32),
        ],
        interpret=interpret,
    )(features, labels_tc)


def _tc_body(x_ref, ps_ref, cft_ref, cnt_ref, cntt_ref, tgt_ref, out_ref):
    x = x_ref[...]
    norm = jnp.sqrt(jnp.sum(x * x, axis=1, keepdims=True))
    x = x / jnp.maximum(norm, 1e-12)
    cf = ps_ref[0] + ps_ref[1] + cft_ref[...]
    s = lax.dot_general(x, cf, dimension_numbers=(((1,), (1,)), ((), ())),
                        preferred_element_type=jnp.float32)
    nums = cnt_ref[pl.ds(0, C_PAD)][None, :] + cntt_ref[0:1, :]
    for i in range(1, NW):
        nums = nums + cnt_ref[pl.ds(i * C_PAD, C_PAD)][None, :]
    mask = (nums > 0).astype(jnp.float32)
    denom = TEMP * (mask * nums + (1.0 - mask))
    sim = s / denom
    exps = jnp.exp(sim) * mask
    sums = jnp.sum(exps, axis=1, keepdims=True) + EPS
    t_row = tgt_ref[...][None, :].astype(jnp.float32)
    t = jnp.transpose(t_row)
    cidx = lax.broadcasted_iota(jnp.int32, (B, C_PAD), 1).astype(jnp.float32)
    onehot = (cidx == t).astype(jnp.float32)
    picked = jnp.sum(exps * onehot, axis=1, keepdims=True)
    logp = jnp.log(picked / sums + EPS)
    out_ref[0, 0] = -jnp.mean(logp)


@functools.partial(jax.jit, static_argnames=("interpret",))
def _tc_loss(x, ps, cft, cnt, cntt, tgt, interpret=False):
    return pl.pallas_call(
        _tc_body,
        out_shape=jax.ShapeDtypeStruct((1, 1), jnp.float32),
        out_specs=pl.BlockSpec(memory_space=pltpu.SMEM),
        interpret=interpret,
    )(x, ps, cft, cnt, cntt, tgt)


def kernel(inputs, indexes, features, labels):
    labels2d = jnp.pad(labels[:SC_ROWS],
                       (0, L2DP * RBLK - SC_ROWS)).reshape(L2DP, RBLK)
    labels_tc = jnp.pad(labels[SC_ROWS:], (0, TC_STEPS * TCB - TC_ROWS),
                        constant_values=-1).reshape(TC_STEPS, 1, TCB)
    zf = jnp.zeros((C_PAD, D), jnp.float32)
    cft, cntt = _tc_segsum(features, labels_tc)
    ps, cnt, targets = _sc_segsum()(features, labels, labels2d,
                                    indexes.astype(jnp.int32), zf)
    out = _tc_loss(inputs, ps, cft, cnt, cntt, targets)
    return out[0, 0]


# speedup vs baseline: 1.1360x; 1.0259x over previous
import functools

import jax
import jax.numpy as jnp
from jax import lax
from jax.experimental import pallas as pl
from jax.experimental.pallas import tpu as pltpu
from jax.experimental.pallas import tpu_sc as plsc

B = 1024
D = 128
NUM_MEMORY = 100000
NUM_CLASSES = 1000
TEMP = 0.05
EPS = 1e-06

C_PAD = 1024
NC, NS = 2, 16
NW = NC * NS
RBLK = 128
CHUNK = 2 * RBLK

SC_ROWS = 80896
NCHUNK = SC_ROWS // CHUNK
EXTRA = NCHUNK % NW
BASEC = NCHUNK // NW
MAXJ = BASEC + 1
TPW = B // NW
CROWS = C_PAD // NS
L2DP = 648
LSTAGE = 32

TC_ROWS = NUM_MEMORY - SC_ROWS
TCB = 1024
TC_STEPS = (TC_ROWS + TCB - 1) // TCB
TC_OFF = SC_ROWS // TCB


def _sc_body(feats, labels_h, labels2d, indexes_h, zf,
             ps_out, cnt_out, tgt_out,
             featA, featB, lab_all, cnt_v, idx_v, tgt_v,
             acc, semA, semB, semS, semT):
    c = lax.axis_index("c")
    s = lax.axis_index("s")
    w = s * NC + c
    schunk = BASEC * w + jnp.minimum(w, EXTRA)
    nchunk = jnp.where(w < EXTRA, BASEC + 1, BASEC)

    pltpu.sync_copy(indexes_h.at[pl.ds(w * TPW, TPW)], idx_v)
    pltpu.async_copy(labels_h.at[idx_v], tgt_v, semT)

    schunk2 = schunk * 2
    base = pl.multiple_of((schunk2 // 8) * 8, 8)
    off = schunk2 - base
    pltpu.sync_copy(labels2d.at[pl.ds(base, LSTAGE)], lab_all)

    pltpu.sync_copy(zf.at[pl.ds(s * CROWS, CROWS)], acc.at[pl.ds(s * CROWS, CROWS)])

    def zero_cnt(i, carry):
        cnt_v[pl.ds(i * 16, 16)] = jnp.zeros((16,), jnp.float32)
        return carry

    lax.fori_loop(0, C_PAD // 16, zero_cnt, 0)
    plsc.subcore_barrier()

    feat_bufs = (featA, featB)
    sems = (semA, semB)
    ones = jnp.full((16,), 1.0, jnp.float32)

    def start(j, b):
        @pl.when(j < nchunk)
        def _():
            st = pl.multiple_of((schunk + j) * CHUNK, CHUNK)
            pltpu.async_copy(feats.at[pl.ds(st, CHUNK)], feat_bufs[b], sems[b])

    def count_row(row):
        for i in range(RBLK // 16):
            lv = lab_all[off + row, pl.ds(i * 16, 16)]
            plsc.addupdate_scatter(cnt_v, [lv], ones)

    def process(j, b):
        start(j + 1, 1 - b)

        @pl.when(j < nchunk)
        def _():
            pltpu.make_async_copy(feats.at[pl.ds(0, CHUNK)],
                                  feat_bufs[b], sems[b]).wait()
            pltpu.async_copy(feat_bufs[b].at[pl.ds(0, RBLK)],
                             acc.at[lab_all.at[off + 2 * j]], semS, add=True)
            pltpu.sync_copy(feat_bufs[b].at[pl.ds(RBLK, RBLK)],
                            acc.at[lab_all.at[off + 2 * j + 1]], add=True)
            pltpu.make_async_copy(feats.at[pl.ds(0, RBLK)],
                                  feat_bufs[b].at[pl.ds(0, RBLK)], semS).wait()
            count_row(2 * j)
            count_row(2 * j + 1)

    start(0, 0)

    def pair_body(j2, carry):
        process(2 * j2, 0)
        process(2 * j2 + 1, 1)
        return carry

    lax.fori_loop(0, (MAXJ + 1) // 2, pair_body, 0)

    pltpu.sync_copy(cnt_v, cnt_out.at[pl.ds(w * C_PAD, C_PAD)])
    pltpu.make_async_copy(labels_h.at[pl.ds(0, TPW)], tgt_v, semT).wait()
    pltpu.sync_copy(tgt_v, tgt_out.at[pl.ds(w * TPW, TPW)])
    plsc.subcore_barrier()
    pltpu.sync_copy(acc.at[pl.ds(s * CROWS, CROWS)],
                    ps_out.at[c, pl.ds(s * CROWS, CROWS)])


@functools.cache
def _sc_segsum():
  return pl.kernel(
    _sc_body,
    out_type=(
        jax.ShapeDtypeStruct((NC, C_PAD, D), jnp.float32),
        jax.ShapeDtypeStruct((NW * C_PAD,), jnp.float32),
        jax.ShapeDtypeStruct((B,), jnp.int32),
    ),
    mesh=plsc.VectorSubcoreMesh(core_axis_name="c", subcore_axis_name="s",
                                num_cores=NC, num_subcores=NS),
    compiler_params=pltpu.CompilerParams(needs_layout_passes=False),
    scratch_types=[
        pltpu.VMEM((CHUNK, D), jnp.float32),
        pltpu.VMEM((CHUNK, D), jnp.float32),
        pltpu.VMEM((LSTAGE, RBLK), jnp.int32),
        pltpu.VMEM((C_PAD,), jnp.float32),
        pltpu.VMEM((TPW,), jnp.int32),
        pltpu.VMEM((TPW,), jnp.int32),
        pltpu.VMEM_SHARED((C_PAD, D), jnp.float32),
        pltpu.SemaphoreType.DMA,
        pltpu.SemaphoreType.DMA,
        pltpu.SemaphoreType.DMA,
        pltpu.SemaphoreType.DMA,
    ],
  )


def _tcseg_body(feat_ref, lab_ref, cf_ref, cnt_ref):
    j = pl.program_id(0)

    @pl.when(j == 0)
    def _():
        cf_ref[...] = jnp.zeros((C_PAD, D), jnp.float32)
        cnt_ref[...] = jnp.zeros((8, C_PAD), jnp.float32)

    lab_row = lab_ref[0, 0:1, :].astype(jnp.float32)
    lab_col = jnp.transpose(lab_row)
    cidx = lax.broadcasted_iota(jnp.int32, (TCB, C_PAD), 1).astype(jnp.float32)
    onehot = (cidx == lab_col).astype(jnp.float32)
    cf_ref[...] += lax.dot_general(onehot, feat_ref[...],
                                   dimension_numbers=(((0,), (0,)), ((), ())),
                                   preferred_element_type=jnp.float32)
    cnt_ref[...] += jnp.broadcast_to(
        jnp.sum(onehot, axis=0, keepdims=True), (8, C_PAD))


@functools.partial(jax.jit, static_argnames=("interpret",))
def _tc_segsum(features, labels_tc, interpret=False):
    return pl.pallas_call(
        _tcseg_body,
        grid=(TC_STEPS,),
        in_specs=[
            pl.BlockSpec((TCB, D), lambda j: (TC_OFF + j, 0)),
            pl.BlockSpec((1, 1, TCB), lambda j: (j, 0, 0)),
        ],
        out_specs=[
            pl.BlockSpec((C_PAD, D), lambda j: (0, 0)),
            pl.BlockSpec((8, C_PAD), lambda j: (0, 0)),
        ],
        out_shape=[
            jax.ShapeDtypeStruct((C_PAD, D), jnp.float32),
            jax.ShapeDtypeStruct((8, C_PAD), jnp.float32),
        ],
        interpret=interpret,
    )(features, labels_tc)


def _tc_body(x_ref, ps_ref, cft_ref, cnt_ref, cntt_ref, tgt_ref, out_ref):
    x = x_ref[...]
    norm = jnp.sqrt(jnp.sum(x * x, axis=1, keepdims=True))
    x = x / jnp.maximum(norm, 1e-12)
    cf = ps_ref[0] + ps_ref[1] + cft_ref[...]
    s = lax.dot_general(x, cf, dimension_numbers=(((1,), (1,)), ((), ())),
                        preferred_element_type=jnp.float32)
    nums = cnt_ref[pl.ds(0, C_PAD)][None, :] + cntt_ref[0:1, :]
    for i in range(1, NW):
        nums = nums + cnt_ref[pl.ds(i * C_PAD, C_PAD)][None, :]
    mask = (nums > 0).astype(jnp.float32)
    denom = TEMP * (mask * nums + (1.0 - mask))
    sim = s / denom
    exps = jnp.exp(sim) * mask
    sums = jnp.sum(exps, axis=1, keepdims=True) + EPS
    t_row = tgt_ref[...][None, :].astype(jnp.float32)
    t = jnp.transpose(t_row)
    cidx = lax.broadcasted_iota(jnp.int32, (B, C_PAD), 1).astype(jnp.float32)
    onehot = (cidx == t).astype(jnp.float32)
    picked = jnp.sum(exps * onehot, axis=1, keepdims=True)
    logp = jnp.log(picked / sums + EPS)
    out_ref[0, 0] = -jnp.mean(logp)


@functools.partial(jax.jit, static_argnames=("interpret",))
def _tc_loss(x, ps, cft, cnt, cntt, tgt, interpret=False):
    return pl.pallas_call(
        _tc_body,
        out_shape=jax.ShapeDtypeStruct((1, 1), jnp.float32),
        out_specs=pl.BlockSpec(memory_space=pltpu.SMEM),
        interpret=interpret,
    )(x, ps, cft, cnt, cntt, tgt)


def kernel(inputs, indexes, features, labels):
    labels2d = jnp.pad(labels[:SC_ROWS],
                       (0, L2DP * RBLK - SC_ROWS)).reshape(L2DP, RBLK)
    labels_tc = jnp.pad(labels[SC_ROWS:], (0, TC_STEPS * TCB - TC_ROWS),
                        constant_values=-1).reshape(TC_STEPS, 1, TCB)
    zf = jnp.zeros((C_PAD, D), jnp.float32)
    cft, cntt = _tc_segsum(features, labels_tc)
    ps, cnt, targets = _sc_segsum()(features, labels, labels2d,
                                    indexes.astype(jnp.int32), zf)
    out = _tc_loss(inputs, ps, cft, cnt, cntt, targets)
    return out[0, 0]
